# baseline (speedup 1.0000x reference)
import functools

import jax
import jax.numpy as jnp
from jax import lax
from jax.experimental import pallas as pl
from jax.experimental.pallas import tpu as pltpu
from jax.experimental.pallas import tpu_sc as plsc

NC = 2
NS = 16
NW = NC * NS
LANES = 16
CH = 128


def _body(idx_hbm, z_hbm, mean_hbm, logvar_hbm, v_hbm, mu_hbm, lv_hbm,
          idx_v, mu_v, lv_v, z_v, sem_mu, sem_lv, sem_z):
    W, L = mean_hbm.shape[1], mean_hbm.shape[2]
    rows = idx_v.shape[0]
    n_chunks = rows // CH
    wid = lax.axis_index("s") * NC + lax.axis_index("c")
    base0 = wid * rows
    pltpu.sync_copy(idx_hbm.at[pl.ds(base0, rows)], idx_v)
    for c in range(n_chunks):
        base = base0 + c * CH
        isl = idx_v.at[pl.ds(c * CH, CH)]
        g_mu = pltpu.async_copy(mean_hbm.at[isl], mu_v, sem_mu)
        g_lv = pltpu.async_copy(logvar_hbm.at[isl], lv_v, sem_lv)
        g_z = pltpu.async_copy(z_hbm.at[pl.ds(base, CH)], z_v, sem_z)
        g_mu.wait()
        g_lv.wait()
        g_z.wait()
        o_mu = pltpu.async_copy(mu_v, mu_hbm.at[pl.ds(base, CH)], sem_mu)
        o_lv = pltpu.async_copy(lv_v, lv_hbm.at[pl.ds(base, CH)], sem_lv)

        def row_body(r, carry):
            for w in range(W):
                for k in range(L // LANES):
                    sl = pl.ds(k * LANES, LANES)
                    z_v[r, w, sl] = (
                        mu_v[r, w, sl]
                        + jnp.exp(lv_v[r, w, sl] * 0.5) * z_v[r, w, sl]
                    )
            return carry

        lax.fori_loop(0, CH, row_body, 0)
        o_mu.wait()
        o_lv.wait()
        pltpu.sync_copy(z_v, v_hbm.at[pl.ds(base, CH)])


@jax.jit
def _sc_embed(i1, z3, mean3, logvar3):
    B = z3.shape[0]
    W, L = mean3.shape[1], mean3.shape[2]
    rows = B // NW
    out = jax.ShapeDtypeStruct((B, W, L), jnp.float32)
    run = functools.partial(
        pl.kernel,
        out_type=[out, out, out],
        mesh=plsc.VectorSubcoreMesh(core_axis_name="c", subcore_axis_name="s"),
        scratch_types=[
            pltpu.VMEM((rows,), jnp.int32),
            pltpu.VMEM((CH, W, L), jnp.float32),
            pltpu.VMEM((CH, W, L), jnp.float32),
            pltpu.VMEM((CH, W, L), jnp.float32),
            pltpu.SemaphoreType.DMA,
            pltpu.SemaphoreType.DMA,
            pltpu.SemaphoreType.DMA,
        ],
        compiler_params=pltpu.CompilerParams(use_tc_tiling_on_sc=False),
    )(_body)
    return run(i1, z3, mean3, logvar3)


def kernel(i, z, mean, logvar):
    return tuple(_sc_embed(i.astype(jnp.int32), z, mean, logvar))
